# baseline (speedup 1.0000x reference)
import functools

import jax
import jax.numpy as jnp
from jax import lax
from jax.experimental import pallas as pl
from jax.experimental.pallas import tpu as pltpu
from jax.experimental.pallas import tpu_sc as plsc

NUM_CORES = 2
NUM_SUBCORES = 16
NW = NUM_CORES * NUM_SUBCORES

CHUNK = 128


def _gather_kernel(b_total, d, keys_hbm, table_hbm, out_hbm, idx_v, rows_v, gsem):
    n_chunks = b_total // (NW * CHUNK)
    wid = lax.axis_index("s") * NUM_CORES + lax.axis_index("c")
    base = wid * n_chunks * CHUNK
    pltpu.sync_copy(keys_hbm.at[wid], idx_v)

    @pl.loop(0, n_chunks)
    def _(j):
        pltpu.async_copy(table_hbm.at[idx_v.at[j]], rows_v, gsem).wait()
        pltpu.sync_copy(rows_v, out_hbm.at[pl.ds(base + j * CHUNK, CHUNK)])


def kernel(keys, table):
    b, f = keys.shape
    v, d = table.shape
    b_total = b * f
    n_chunks = b_total // (NW * CHUNK)
    keys_flat = keys.reshape(NW, n_chunks, CHUNK)

    mesh = plsc.VectorSubcoreMesh(core_axis_name="c", subcore_axis_name="s")
    out = pl.kernel(
        functools.partial(_gather_kernel, b_total, d),
        out_type=jax.ShapeDtypeStruct((b_total, d), table.dtype),
        mesh=mesh,
        scratch_types=[
            pltpu.VMEM((n_chunks, CHUNK), jnp.int32),
            pltpu.VMEM((CHUNK, d), jnp.float32),
            pltpu.SemaphoreType.DMA,
        ],
        compiler_params=pltpu.CompilerParams(use_tc_tiling_on_sc=False),
    )(keys_flat, table)
    return out.reshape(b, f, d)


# speedup vs baseline: 1.0788x; 1.0788x over previous
import functools

import jax
import jax.numpy as jnp
from jax import lax
from jax.experimental import pallas as pl
from jax.experimental.pallas import tpu as pltpu
from jax.experimental.pallas import tpu_sc as plsc

NUM_CORES = 2
NUM_SUBCORES = 16
NW = NUM_CORES * NUM_SUBCORES

CHUNK = 128
NBUF = 4
NB2 = 2 * NBUF


def _gather_kernel(n_chunks, keys_hbm, table_hbm, out_hbm, idx_v, rows_v,
                   gsem, wsem):
    wid = lax.axis_index("s") * NUM_CORES + lax.axis_index("c")
    base = wid * n_chunks * CHUNK
    pltpu.sync_copy(keys_hbm.at[wid], idx_v)

    def start_gather(c, b):
        pltpu.async_copy(table_hbm.at[idx_v.at[c]], rows_v.at[b], gsem.at[b])

    def wait_gather(c, b):
        pltpu.make_async_copy(
            table_hbm.at[idx_v.at[c]], rows_v.at[b], gsem.at[b]).wait()

    def start_write(c, b):
        pltpu.async_copy(
            rows_v.at[b], out_hbm.at[pl.ds(base + c * CHUNK, CHUNK)],
            wsem.at[b])

    def wait_write(c, b):
        pltpu.make_async_copy(
            rows_v.at[b], out_hbm.at[pl.ds(base + c * CHUNK, CHUNK)],
            wsem.at[b]).wait()

    for b in range(NBUF):
        start_gather(b, b)

    for c in range(NBUF):
        wait_gather(c, c)
        start_write(c, c)
        start_gather(c + NBUF, c + NBUF)

    @pl.loop(NBUF, n_chunks - NBUF, step=NB2)
    def _(i):
        for k in range(NB2):
            c = i + k
            b = (NBUF + k) % NB2
            bn = (b + NBUF) % NB2
            wait_gather(c, b)
            start_write(c, b)
            wait_write(c - NBUF, bn)
            start_gather(c + NBUF, bn)

    for k in range(NBUF):
        c = n_chunks - NBUF + k
        b = c % NB2
        wait_gather(c, b)
        start_write(c, b)

    for j in range(NB2):
        c = n_chunks - NB2 + j
        wait_write(c, c % NB2)


def kernel(keys, table):
    b, f = keys.shape
    v, d = table.shape
    b_total = b * f
    n_chunks = b_total // (NW * CHUNK)
    keys_flat = keys.reshape(NW, n_chunks, CHUNK)

    mesh = plsc.VectorSubcoreMesh(core_axis_name="c", subcore_axis_name="s")
    out = pl.kernel(
        functools.partial(_gather_kernel, n_chunks),
        out_type=jax.ShapeDtypeStruct((b_total, d), table.dtype),
        mesh=mesh,
        scratch_types=[
            pltpu.VMEM((n_chunks, CHUNK), jnp.int32),
            pltpu.VMEM((NB2, CHUNK, d), jnp.float32),
            pltpu.SemaphoreType.DMA((NB2,)),
            pltpu.SemaphoreType.DMA((NB2,)),
        ],
        compiler_params=pltpu.CompilerParams(use_tc_tiling_on_sc=False),
    )(keys_flat, table)
    return out.reshape(b, f, d)
